# Initial kernel scaffold; baseline (speedup 1.0000x reference)
import jax
import jax.numpy as jnp
from jax.experimental import pallas as pl


def kernel(input, nodes, edge_sources, edge_targets, rij, combine_sets, plane_wave, cutoff, W1, W2, W2g, Wg, Wm):
    raise NotImplementedError("write your pallas kernel here")


# speedup vs baseline: 2.9303x; 2.9303x over previous
import functools

import jax
import jax.numpy as jnp
from jax import lax
from jax.experimental import pallas as pl
from jax.experimental.pallas import tpu as pltpu
from jax.experimental.pallas import tpu_sc as plsc

N = 10000
E = 320000
D = 128
K = 64
INF = 3 * D

NC = 2
NS = 16
NW = NC * NS
C = 128
NCHUNKS = E // C
N_ACC = 10240
ROWS_PER_SUB = N_ACC // NS

_sc_mesh = plsc.VectorSubcoreMesh(core_axis_name="c", subcore_axis_name="s")


def _worker_id():
    return lax.axis_index("s") * NC + lax.axis_index("c")


def _chunk_loop(wid, body):
    base_chunks = NCHUNKS // NW
    extra = NCHUNKS - base_chunks * NW
    n = jnp.where(wid < extra, base_chunks + 1, base_chunks)

    def step(j, _):
        body((wid + j * NW) * C)
        return 0

    lax.fori_loop(0, n, step, 0)


@functools.partial(
    pl.kernel,
    out_type=(
        jax.ShapeDtypeStruct((E, D), jnp.float32),
        jax.ShapeDtypeStruct((E, D), jnp.float32),
    ),
    mesh=_sc_mesh,
    scratch_types=(
        pltpu.VMEM((C,), jnp.int32),
        pltpu.VMEM((C,), jnp.int32),
        pltpu.VMEM((C, D), jnp.float32),
        pltpu.VMEM((C, D), jnp.float32),
        pltpu.SemaphoreType.DMA,
        pltpu.SemaphoreType.DMA,
    ),
)
def _gather(x_hbm, src_hbm, tgt_hbm, ni_hbm, nj_hbm,
            idx_s, idx_t, rows_s, rows_t, sem_s, sem_t):
    wid = _worker_id()

    def body(base):
        pltpu.sync_copy(src_hbm.at[pl.ds(base, C)], idx_s)
        pltpu.sync_copy(tgt_hbm.at[pl.ds(base, C)], idx_t)
        a = pltpu.async_copy(x_hbm.at[idx_s], rows_s, sem_s)
        b = pltpu.async_copy(x_hbm.at[idx_t], rows_t, sem_t)
        a.wait()
        b.wait()
        pltpu.sync_copy(rows_s, ni_hbm.at[pl.ds(base, C)])
        pltpu.sync_copy(rows_t, nj_hbm.at[pl.ds(base, C)])

    _chunk_loop(wid, body)


@functools.partial(
    pl.kernel,
    out_type=jax.ShapeDtypeStruct((NC, N_ACC, D), jnp.float32),
    mesh=_sc_mesh,
    scratch_types=(
        pltpu.VMEM_SHARED((N_ACC, D), jnp.float32),
        pltpu.VMEM((C,), jnp.int32),
        pltpu.VMEM((C, D), jnp.float32),
    ),
)
def _scatter(z_hbm, src_hbm, zero_hbm, part_hbm, acc, idx_v, rows_v):
    cid = lax.axis_index("c")
    sid = lax.axis_index("s")
    wid = sid * NC + cid

    pltpu.sync_copy(zero_hbm.at[pl.ds(sid * ROWS_PER_SUB, ROWS_PER_SUB)],
                    acc.at[pl.ds(sid * ROWS_PER_SUB, ROWS_PER_SUB)])
    plsc.subcore_barrier()

    def body(base):
        pltpu.sync_copy(src_hbm.at[pl.ds(base, C)], idx_v)
        pltpu.sync_copy(z_hbm.at[pl.ds(base, C)], rows_v)
        pltpu.sync_copy(rows_v, acc.at[idx_v], add=True)

    _chunk_loop(wid, body)

    plsc.subcore_barrier()
    pltpu.sync_copy(acc.at[pl.ds(sid * ROWS_PER_SUB, ROWS_PER_SUB)],
                    part_hbm.at[cid].at[pl.ds(sid * ROWS_PER_SUB, ROWS_PER_SUB)])


BE = 2000


def _edge_mlp_body(rij_ref, cut_ref, cs_ref, pw_ref, ni_ref, nj_ref,
                   w1_ref, w2_ref, w2g_ref, wg_ref, wm_ref, z_ref):
    ni = ni_ref[...]
    nj = nj_ref[...]
    r = rij_ref[...]
    delta = (ni - nj) / r
    fe = jnp.concatenate([ni, nj, delta], axis=1)
    g = jnp.dot(fe, wg_ref[...], preferred_element_type=jnp.float32)
    g = 1.0 / (1.0 + jnp.exp(-g))
    m = jnp.dot(fe, wm_ref[...], preferred_element_type=jnp.float32)
    m = jnp.where(m > 0, m, jnp.exp(jnp.minimum(m, 0.0)) - 1.0)
    pw = pw_ref[...]
    gate = jnp.dot(pw, w2g_ref[...], preferred_element_type=jnp.float32)
    gate = 1.0 / (1.0 + jnp.exp(-gate))
    z2 = jnp.dot(pw * gate, w2_ref[...], preferred_element_type=jnp.float32)
    z1 = jnp.dot(cs_ref[...], w1_ref[...], preferred_element_type=jnp.float32)
    mask = (r < cut_ref[0]).astype(jnp.float32)
    z_ref[...] = g * m * (z1 + z2) * mask


def _edge_mlp(rij2, cutoff, cs, pw, ni, nj, w1, w2, w2g, wg, wm):
    grid = (E // BE,)
    full = lambda shape: pl.BlockSpec(shape, lambda i: (0,) * len(shape))
    return pl.pallas_call(
        _edge_mlp_body,
        grid=grid,
        in_specs=[
            pl.BlockSpec((BE, 1), lambda i: (i, 0)),
            pl.BlockSpec(memory_space=pltpu.SMEM),
            pl.BlockSpec((BE, K), lambda i: (i, 0)),
            pl.BlockSpec((BE, K), lambda i: (i, 0)),
            pl.BlockSpec((BE, D), lambda i: (i, 0)),
            pl.BlockSpec((BE, D), lambda i: (i, 0)),
            full((K, D)),
            full((K, D)),
            full((K, K)),
            full((INF, D)),
            full((INF, D)),
        ],
        out_specs=pl.BlockSpec((BE, D), lambda i: (i, 0)),
        out_shape=jax.ShapeDtypeStruct((E, D), jnp.float32),
    )(rij2, cutoff, cs, pw, ni, nj, w1, w2, w2g, wg, wm)


BN = 1000


def _combine_body(x_ref, p_ref, o_ref):
    o_ref[...] = x_ref[...] + p_ref[0] + p_ref[1]


def _combine(x, parts):
    return pl.pallas_call(
        _combine_body,
        grid=(N // BN,),
        in_specs=[
            pl.BlockSpec((BN, D), lambda i: (i, 0)),
            pl.BlockSpec((NC, BN, D), lambda i: (0, i, 0)),
        ],
        out_specs=pl.BlockSpec((BN, D), lambda i: (i, 0)),
        out_shape=jax.ShapeDtypeStruct((N, D), jnp.float32),
    )(x, parts)


def kernel(input, nodes, edge_sources, edge_targets, rij, combine_sets,
           plane_wave, cutoff, W1, W2, W2g, Wg, Wm):
    ni, nj = _gather(input, edge_sources, edge_targets)
    z = _edge_mlp(rij[:, None], cutoff, combine_sets, plane_wave, ni, nj,
                  W1, W2, W2g, Wg, Wm)
    zero = jnp.zeros((N_ACC, D), jnp.float32)
    parts = _scatter(z, edge_sources, zero)
    return _combine(input, parts)


# speedup vs baseline: 3.1157x; 1.0633x over previous
import functools

import jax
import jax.numpy as jnp
from jax import lax
from jax.experimental import pallas as pl
from jax.experimental.pallas import tpu as pltpu
from jax.experimental.pallas import tpu_sc as plsc

N = 10000
E = 320000
D = 128
K = 64
INF = 3 * D

NC = 2
NS = 16
NW = NC * NS
PER_W = E // NW
C = 128
NFULL = PER_W // C
TAIL = PER_W - NFULL * C
N_ACC = 10240
ROWS_PER_SUB = N_ACC // NS

_sc_mesh = plsc.VectorSubcoreMesh(core_axis_name="c", subcore_axis_name="s")


def _worker_id():
    return lax.axis_index("s") * NC + lax.axis_index("c")


@functools.partial(
    pl.kernel,
    out_type=(
        jax.ShapeDtypeStruct((E, D), jnp.float32),
        jax.ShapeDtypeStruct((E, D), jnp.float32),
    ),
    mesh=_sc_mesh,
    scratch_types=(
        pltpu.VMEM((2, C), jnp.int32),
        pltpu.VMEM((2, C), jnp.int32),
        pltpu.VMEM((2, C, D), jnp.float32),
        pltpu.VMEM((2, C, D), jnp.float32),
        pltpu.SemaphoreType.DMA,
        pltpu.SemaphoreType.DMA,
        pltpu.VMEM((TAIL,), jnp.int32),
        pltpu.VMEM((TAIL,), jnp.int32),
        pltpu.VMEM((TAIL, D), jnp.float32),
        pltpu.VMEM((TAIL, D), jnp.float32),
    ),
)
def _gather(x_hbm, src_hbm, tgt_hbm, ni_hbm, nj_hbm,
            idx_s, idx_t, rows_s, rows_t, sem0, sem1,
            idx_s3, idx_t3, rows_s3, rows_t3):
    wid = _worker_id()
    base_w = wid * PER_W

    def fire(j, b, sem):
        base = base_w + j * C
        pltpu.sync_copy(src_hbm.at[pl.ds(base, C)], idx_s.at[b])
        pltpu.sync_copy(tgt_hbm.at[pl.ds(base, C)], idx_t.at[b])
        pltpu.async_copy(x_hbm.at[idx_s.at[b]], rows_s.at[b], sem)
        pltpu.async_copy(x_hbm.at[idx_t.at[b]], rows_t.at[b], sem)

    def drain_write(j, b, sem):
        base = base_w + j * C
        pltpu.make_async_copy(x_hbm.at[idx_s.at[b]], rows_s.at[b], sem).wait()
        pltpu.make_async_copy(x_hbm.at[idx_t.at[b]], rows_t.at[b], sem).wait()
        pltpu.sync_copy(rows_s.at[b], ni_hbm.at[pl.ds(base, C)])
        pltpu.sync_copy(rows_t.at[b], nj_hbm.at[pl.ds(base, C)])

    fire(0, 0, sem0)

    def g_loop(g, _):
        fire(2 * g + 1, 1, sem1)
        drain_write(2 * g, 0, sem0)

        @pl.when(2 * g + 2 < NFULL)
        def _():
            fire(2 * g + 2, 0, sem0)

        drain_write(2 * g + 1, 1, sem1)
        return 0

    lax.fori_loop(0, NFULL // 2, g_loop, 0)

    base = base_w + NFULL * C
    pltpu.sync_copy(src_hbm.at[pl.ds(base, TAIL)], idx_s3)
    pltpu.sync_copy(tgt_hbm.at[pl.ds(base, TAIL)], idx_t3)
    a = pltpu.async_copy(x_hbm.at[idx_s3], rows_s3, sem0)
    b = pltpu.async_copy(x_hbm.at[idx_t3], rows_t3, sem1)
    a.wait()
    b.wait()
    pltpu.sync_copy(rows_s3, ni_hbm.at[pl.ds(base, TAIL)])
    pltpu.sync_copy(rows_t3, nj_hbm.at[pl.ds(base, TAIL)])


@functools.partial(
    pl.kernel,
    out_type=jax.ShapeDtypeStruct((NC, N_ACC, D), jnp.float32),
    mesh=_sc_mesh,
    scratch_types=(
        pltpu.VMEM_SHARED((N_ACC, D), jnp.float32),
        pltpu.VMEM((2, C), jnp.int32),
        pltpu.VMEM((2, C, D), jnp.float32),
        pltpu.SemaphoreType.DMA,
        pltpu.SemaphoreType.DMA,
        pltpu.VMEM((TAIL,), jnp.int32),
        pltpu.VMEM((TAIL, D), jnp.float32),
    ),
)
def _scatter(z_hbm, src_hbm, zero_hbm, part_hbm,
             acc, idx_v, rows_v, sem0, sem1, idx_v3, rows_v3):
    cid = lax.axis_index("c")
    sid = lax.axis_index("s")
    wid = _worker_id()
    base_w = wid * PER_W

    pltpu.sync_copy(zero_hbm.at[pl.ds(sid * ROWS_PER_SUB, ROWS_PER_SUB)],
                    acc.at[pl.ds(sid * ROWS_PER_SUB, ROWS_PER_SUB)])
    plsc.subcore_barrier()

    def fire(j, b, sem):
        base = base_w + j * C
        pltpu.async_copy(src_hbm.at[pl.ds(base, C)], idx_v.at[b], sem)
        pltpu.async_copy(z_hbm.at[pl.ds(base, C)], rows_v.at[b], sem)

    def drain_add(j, b, sem):
        base = base_w + j * C
        pltpu.make_async_copy(src_hbm.at[pl.ds(base, C)], idx_v.at[b], sem).wait()
        pltpu.make_async_copy(z_hbm.at[pl.ds(base, C)], rows_v.at[b], sem).wait()
        pltpu.sync_copy(rows_v.at[b], acc.at[idx_v.at[b]], add=True)

    fire(0, 0, sem0)

    def g_loop(g, _):
        fire(2 * g + 1, 1, sem1)
        drain_add(2 * g, 0, sem0)

        @pl.when(2 * g + 2 < NFULL)
        def _():
            fire(2 * g + 2, 0, sem0)

        drain_add(2 * g + 1, 1, sem1)
        return 0

    lax.fori_loop(0, NFULL // 2, g_loop, 0)

    base = base_w + NFULL * C
    pltpu.sync_copy(src_hbm.at[pl.ds(base, TAIL)], idx_v3)
    pltpu.sync_copy(z_hbm.at[pl.ds(base, TAIL)], rows_v3)
    pltpu.sync_copy(rows_v3, acc.at[idx_v3], add=True)

    plsc.subcore_barrier()
    pltpu.sync_copy(acc.at[pl.ds(sid * ROWS_PER_SUB, ROWS_PER_SUB)],
                    part_hbm.at[cid].at[pl.ds(sid * ROWS_PER_SUB, ROWS_PER_SUB)])


BE = 2000


def _edge_mlp_body(rij_ref, cut_ref, cs_ref, pw_ref, ni_ref, nj_ref,
                   w1_ref, w2_ref, w2g_ref, wg_ref, wm_ref, z_ref):
    ni = ni_ref[...]
    nj = nj_ref[...]
    r = rij_ref[...]
    delta = (ni - nj) / r
    fe = jnp.concatenate([ni, nj, delta], axis=1).astype(jnp.bfloat16)
    g = jnp.dot(fe, wg_ref[...], preferred_element_type=jnp.float32)
    g = 1.0 / (1.0 + jnp.exp(-g))
    m = jnp.dot(fe, wm_ref[...], preferred_element_type=jnp.float32)
    m = jnp.where(m > 0, m, jnp.exp(jnp.minimum(m, 0.0)) - 1.0)
    pw = pw_ref[...]
    gate = jnp.dot(pw, w2g_ref[...], preferred_element_type=jnp.float32)
    gate = 1.0 / (1.0 + jnp.exp(-gate))
    z2 = jnp.dot(pw * gate, w2_ref[...], preferred_element_type=jnp.float32)
    z1 = jnp.dot(cs_ref[...], w1_ref[...], preferred_element_type=jnp.float32)
    mask = (r < cut_ref[0]).astype(jnp.float32)
    z_ref[...] = g * m * (z1 + z2) * mask


def _edge_mlp(rij2, cutoff, cs, pw, ni, nj, w1, w2, w2g, wg, wm):
    grid = (E // BE,)
    full = lambda shape: pl.BlockSpec(shape, lambda i: (0,) * len(shape))
    return pl.pallas_call(
        _edge_mlp_body,
        grid=grid,
        in_specs=[
            pl.BlockSpec((BE, 1), lambda i: (i, 0)),
            pl.BlockSpec(memory_space=pltpu.SMEM),
            pl.BlockSpec((BE, K), lambda i: (i, 0)),
            pl.BlockSpec((BE, K), lambda i: (i, 0)),
            pl.BlockSpec((BE, D), lambda i: (i, 0)),
            pl.BlockSpec((BE, D), lambda i: (i, 0)),
            full((K, D)),
            full((K, D)),
            full((K, K)),
            full((INF, D)),
            full((INF, D)),
        ],
        out_specs=pl.BlockSpec((BE, D), lambda i: (i, 0)),
        out_shape=jax.ShapeDtypeStruct((E, D), jnp.float32),
    )(rij2, cutoff, cs, pw, ni, nj, w1, w2, w2g, wg, wm)


BN = 1000


def _combine_body(x_ref, p_ref, o_ref):
    o_ref[...] = x_ref[...] + p_ref[0] + p_ref[1]


def _combine(x, parts):
    return pl.pallas_call(
        _combine_body,
        grid=(N // BN,),
        in_specs=[
            pl.BlockSpec((BN, D), lambda i: (i, 0)),
            pl.BlockSpec((NC, BN, D), lambda i: (0, i, 0)),
        ],
        out_specs=pl.BlockSpec((BN, D), lambda i: (i, 0)),
        out_shape=jax.ShapeDtypeStruct((N, D), jnp.float32),
    )(x, parts)


def kernel(input, nodes, edge_sources, edge_targets, rij, combine_sets,
           plane_wave, cutoff, W1, W2, W2g, Wg, Wm):
    ni, nj = _gather(input, edge_sources, edge_targets)
    z = _edge_mlp(rij[:, None], cutoff, combine_sets, plane_wave, ni, nj,
                  W1, W2, W2g,
                  Wg.astype(jnp.bfloat16), Wm.astype(jnp.bfloat16))
    zero = jnp.zeros((N_ACC, D), jnp.float32)
    parts = _scatter(z, edge_sources, zero)
    return _combine(input, parts)
